# baseline (speedup 1.0000x reference)
import functools

import jax
import jax.numpy as jnp
from jax import lax
from jax.experimental import pallas as pl
from jax.experimental.pallas import tpu as pltpu
from jax.experimental.pallas import tpu_sc as plsc

Q = 1024
D = 128
KB = 2048


def _normalize_bf16(x):
    return (x / (jnp.sqrt(jnp.sum(x * x, axis=1, keepdims=True)) + 1e-12)
            ).astype(jnp.bfloat16)


def _pen_from(lab):
    return jnp.where(lab > 0, jnp.float32(1.0), jnp.float32(jnp.inf))


def _argmin_body(q_ref, k_ref, lab_ref, kt_ref, labt_ref, idx_ref,
                 qn_ref, kna_ref, knb_ref, sa_ref, sb_ref,
                 pena_ref, penb_ref, rmin_ref, rjdx_ref):
    t = pl.program_id(0)
    nsteps = pl.num_programs(0)
    nblk = nsteps - 2
    even = lax.rem(t, 2) == 0

    @pl.when(t == 0)
    def _init():
        qn_ref[...] = _normalize_bf16(q_ref[...])
        rmin_ref[...] = jnp.full_like(rmin_ref, jnp.inf)
        rjdx_ref[...] = jnp.zeros_like(rjdx_ref)

    def _elem(s_ref, pen_ref):
        dd = pen_ref[...] - s_ref[...]
        upd = dd < rmin_ref[...]
        rjdx_ref[...] = jnp.where(upd, t - 2, rjdx_ref[...])
        rmin_ref[...] = jnp.where(upd, dd, rmin_ref[...])

    @pl.when((t >= 2) & even)
    def _e_a():
        _elem(sa_ref, pena_ref)

    @pl.when((t >= 2) & jnp.logical_not(even))
    def _e_b():
        _elem(sb_ref, penb_ref)

    def _norm(kn_ref, pen_ref):
        kn_ref[...] = _normalize_bf16(k_ref[...])
        pen_ref[...] = _pen_from(lab_ref[...].reshape(1, KB))

    @pl.when((t < nblk) & even)
    def _n_a():
        _norm(kna_ref, pena_ref)

    @pl.when((t < nblk) & jnp.logical_not(even))
    def _n_b():
        _norm(knb_ref, penb_ref)

    def _mm(kn_ref, s_ref):
        s_ref[...] = lax.dot_general(
            qn_ref[...], kn_ref[...],
            dimension_numbers=(((1,), (1,)), ((), ())),
            preferred_element_type=jnp.float32,
        )

    @pl.when((t >= 1) & (t <= nblk) & even)
    def _m_b():
        _mm(knb_ref, sb_ref)

    @pl.when((t >= 1) & (t <= nblk) & jnp.logical_not(even))
    def _m_a():
        _mm(kna_ref, sa_ref)

    @pl.when(t == nsteps - 1)
    def _final():
        nmain = nblk * KB
        knt = _normalize_bf16(kt_ref[...])
        st = lax.dot_general(
            qn_ref[...], knt,
            dimension_numbers=(((1,), (1,)), ((), ())),
            preferred_element_type=jnp.float32,
        )
        ddt = _pen_from(labt_ref[...]) - st
        tmin = jnp.min(ddt, axis=1, keepdims=True)
        targ = (jnp.argmin(ddt, axis=1).astype(jnp.int32).reshape(Q, 1)
                + nmain)

        rmin = rmin_ref[...]
        m = jnp.min(rmin, axis=1, keepdims=True)
        c = lax.broadcasted_iota(jnp.int32, (Q, KB), 1)
        cand = jnp.where(rmin == m, rjdx_ref[...] * KB + c,
                         jnp.int32(2**31 - 1))
        idx_main = jnp.min(cand, axis=1, keepdims=True)
        idx_ref[...] = jnp.where(tmin < m, targ, idx_main)


def _nearest_index(queries, keys, label_confi):
    K = keys.shape[0]
    nblk = K // KB
    nmain = nblk * KB
    kt = K - nmain
    lab = label_confi.astype(jnp.int32)
    lab3d = lab[:nmain].reshape(nblk, 1, KB)
    keys_tail = keys[nmain:]
    lab_tail = lab[nmain:].reshape(1, kt)
    clamp = lambda j: jnp.minimum(j, nblk - 1)
    idx = pl.pallas_call(
        _argmin_body,
        grid=(nblk + 2,),
        in_specs=[
            pl.BlockSpec((Q, D), lambda j: (0, 0)),
            pl.BlockSpec((KB, D), lambda j: (clamp(j), 0)),
            pl.BlockSpec((1, 1, KB), lambda j: (clamp(j), 0, 0)),
            pl.BlockSpec((kt, D), lambda j: (0, 0)),
            pl.BlockSpec((1, kt), lambda j: (0, 0)),
        ],
        out_specs=pl.BlockSpec((Q, 1), lambda j: (0, 0)),
        out_shape=jax.ShapeDtypeStruct((Q, 1), jnp.int32),
        scratch_shapes=[
            pltpu.VMEM((Q, D), jnp.bfloat16),
            pltpu.VMEM((KB, D), jnp.bfloat16),
            pltpu.VMEM((KB, D), jnp.bfloat16),
            pltpu.VMEM((Q, KB), jnp.float32),
            pltpu.VMEM((Q, KB), jnp.float32),
            pltpu.VMEM((1, KB), jnp.float32),
            pltpu.VMEM((1, KB), jnp.float32),
            pltpu.VMEM((Q, KB), jnp.float32),
            pltpu.VMEM((Q, KB), jnp.int32),
        ],
    )(queries, keys, lab3d, keys_tail, lab_tail)
    return idx.reshape(Q)


def _make_sc_gather(V, B, Dm):
    NC, NS = 2, 16
    NW = NC * NS
    b_per_w = B // NW
    mesh = plsc.VectorSubcoreMesh(core_axis_name="c", subcore_axis_name="s")

    @functools.partial(
        pl.kernel,
        mesh=mesh,
        out_type=jax.ShapeDtypeStruct((B, Dm), jnp.float32),
        scratch_types=[
            pltpu.VMEM((b_per_w,), jnp.int32),
            pltpu.VMEM((b_per_w, Dm), jnp.float32),
            pltpu.SemaphoreType.DMA,
        ],
    )
    def gather_rows(idx_hbm, table_hbm, out_hbm, idx_v, rows_v, sem):
        wid = lax.axis_index("s") * NC + lax.axis_index("c")
        base = wid * b_per_w
        pltpu.sync_copy(idx_hbm.at[pl.ds(base, b_per_w)], idx_v)
        pltpu.async_copy(table_hbm.at[idx_v], rows_v, sem).wait()
        pltpu.sync_copy(rows_v, out_hbm.at[pl.ds(base, b_per_w)])

    return gather_rows


def kernel(queries, keys, label_confi):
    nearest_idx = _nearest_index(queries, keys, label_confi)
    gather = _make_sc_gather(keys.shape[0], Q, D)
    return gather(nearest_idx, keys)


# speedup vs baseline: 1.7470x; 1.7470x over previous
import functools

import jax
import jax.numpy as jnp
from jax import lax
from jax.experimental import pallas as pl
from jax.experimental.pallas import tpu as pltpu
from jax.experimental.pallas import tpu_sc as plsc

Q = 1024
D = 128
KB = 2048


def _normalize_bf16(x):
    return (x / (jnp.sqrt(jnp.sum(x * x, axis=1, keepdims=True)) + 1e-12)
            ).astype(jnp.bfloat16)


def _pen_from(lab):
    return jnp.where(lab > 0, jnp.float32(1.0), jnp.float32(jnp.inf))


def _argmin_body(q_ref, k_ref, labe_ref, kt_ref, labt_ref, idx_ref,
                 qn_ref, kn_ref, s_ref, rmin_ref, rjdx_ref):
    t = pl.program_id(0)
    nsteps = pl.num_programs(0)
    nblk = nsteps - 2

    @pl.when(t == 0)
    def _init():
        qn_ref[...] = _normalize_bf16(q_ref[...])
        rmin_ref[...] = jnp.full_like(rmin_ref, jnp.inf)
        rjdx_ref[...] = jnp.zeros_like(rjdx_ref)


    pen = _pen_from(labe_ref[...].reshape(1, KB))
    dd = pen - s_ref[...]
    upd = dd < rmin_ref[...]
    rjdx_ref[...] = jnp.where(upd, t - 2, rjdx_ref[...])
    rmin_ref[...] = jnp.where(upd, dd, rmin_ref[...])

    s_ref[...] = lax.dot_general(
        qn_ref[...], kn_ref[...],
        dimension_numbers=(((1,), (1,)), ((), ())),
        preferred_element_type=jnp.float32,
    )

    kn_ref[...] = _normalize_bf16(k_ref[...])

    @pl.when(t == nsteps - 1)
    def _final():
        nmain = nblk * KB
        knt = _normalize_bf16(kt_ref[...])
        st = lax.dot_general(
            qn_ref[...], knt,
            dimension_numbers=(((1,), (1,)), ((), ())),
            preferred_element_type=jnp.float32,
        )
        ddt = _pen_from(labt_ref[...]) - st
        tmin = jnp.min(ddt, axis=1, keepdims=True)
        targ = (jnp.argmin(ddt, axis=1).astype(jnp.int32).reshape(Q, 1)
                + nmain)

        rmin = rmin_ref[...]
        m = jnp.min(rmin, axis=1, keepdims=True)
        c = lax.broadcasted_iota(jnp.int32, (Q, KB), 1)
        cand = jnp.where(rmin == m, rjdx_ref[...] * KB + c,
                         jnp.int32(2**31 - 1))
        idx_main = jnp.min(cand, axis=1, keepdims=True)
        idx_ref[...] = jnp.where(tmin < m, targ, idx_main)


def _nearest_index(queries, keys, label_confi):
    K = keys.shape[0]
    nblk = K // KB
    nmain = nblk * KB
    kt = K - nmain
    lab = label_confi.astype(jnp.int32)
    lab3d = lab[:nmain].reshape(nblk, 1, KB)
    labe = jnp.pad(lab3d, ((1, 0), (0, 0), (0, 0)))
    keys_tail = keys[nmain:]
    lab_tail = lab[nmain:].reshape(1, kt)
    clamp = lambda j: jnp.minimum(j, nblk - 1)
    idx = pl.pallas_call(
        _argmin_body,
        grid=(nblk + 2,),
        in_specs=[
            pl.BlockSpec((Q, D), lambda j: (0, 0)),
            pl.BlockSpec((KB, D), lambda j: (clamp(j), 0)),
            pl.BlockSpec((1, 1, KB), lambda j: (jnp.maximum(j - 1, 0), 0, 0)),
            pl.BlockSpec((kt, D), lambda j: (0, 0)),
            pl.BlockSpec((1, kt), lambda j: (0, 0)),
        ],
        out_specs=pl.BlockSpec((Q, 1), lambda j: (0, 0)),
        out_shape=jax.ShapeDtypeStruct((Q, 1), jnp.int32),
        scratch_shapes=[
            pltpu.VMEM((Q, D), jnp.bfloat16),
            pltpu.VMEM((KB, D), jnp.bfloat16),
            pltpu.VMEM((Q, KB), jnp.float32),
            pltpu.VMEM((Q, KB), jnp.float32),
            pltpu.VMEM((Q, KB), jnp.int32),
        ],
    )(queries, keys, labe, keys_tail, lab_tail)
    return idx.reshape(Q)


def _make_sc_gather(V, B, Dm):
    NC, NS = 2, 16
    NW = NC * NS
    b_per_w = B // NW
    mesh = plsc.VectorSubcoreMesh(core_axis_name="c", subcore_axis_name="s")

    @functools.partial(
        pl.kernel,
        mesh=mesh,
        out_type=jax.ShapeDtypeStruct((B, Dm), jnp.float32),
        scratch_types=[
            pltpu.VMEM((b_per_w,), jnp.int32),
            pltpu.VMEM((b_per_w, Dm), jnp.float32),
            pltpu.SemaphoreType.DMA,
        ],
    )
    def gather_rows(idx_hbm, table_hbm, out_hbm, idx_v, rows_v, sem):
        wid = lax.axis_index("s") * NC + lax.axis_index("c")
        base = wid * b_per_w
        pltpu.sync_copy(idx_hbm.at[pl.ds(base, b_per_w)], idx_v)
        pltpu.async_copy(table_hbm.at[idx_v], rows_v, sem).wait()
        pltpu.sync_copy(rows_v, out_hbm.at[pl.ds(base, b_per_w)])

    return gather_rows


def kernel(queries, keys, label_confi):
    nearest_idx = _nearest_index(queries, keys, label_confi)
    gather = _make_sc_gather(keys.shape[0], Q, D)
    return gather(nearest_idx, keys)


# speedup vs baseline: 1.8829x; 1.0778x over previous
import functools

import jax
import jax.numpy as jnp
from jax import lax
from jax.experimental import pallas as pl
from jax.experimental.pallas import tpu as pltpu
from jax.experimental.pallas import tpu_sc as plsc

Q = 1024
D = 128
KB = 2000


def _argmin_body(q_ref, k_ref, lab_ref, idx_ref, qn_ref, rmin_ref, rjdx_ref):
    j = pl.program_id(0)
    nblk = pl.num_programs(0)

    @pl.when(j == 0)
    def _init():
        q = q_ref[...]
        qn_ref[...] = (
            q / (jnp.sqrt(jnp.sum(q * q, axis=1, keepdims=True)) + 1e-12)
        ).astype(jnp.bfloat16)
        rmin_ref[...] = jnp.full_like(rmin_ref, jnp.inf)
        rjdx_ref[...] = jnp.zeros_like(rjdx_ref)

    k = k_ref[...]
    kn = (k / (jnp.sqrt(jnp.sum(k * k, axis=1, keepdims=True)) + 1e-12)
          ).astype(jnp.bfloat16)

    s = lax.dot_general(
        qn_ref[...], kn,
        dimension_numbers=(((1,), (1,)), ((), ())),
        preferred_element_type=jnp.float32,
    )
    lab = lab_ref[...].reshape(1, KB)
    pen = jnp.where(lab > 0, jnp.float32(1.0), jnp.float32(jnp.inf))
    dd = pen - s

    upd = dd < rmin_ref[...]
    rjdx_ref[...] = jnp.where(upd, j, rjdx_ref[...])
    rmin_ref[...] = jnp.minimum(rmin_ref[...], dd)

    @pl.when(j == nblk - 1)
    def _final():
        rmin = rmin_ref[...]
        m = jnp.min(rmin, axis=1, keepdims=True)
        c = lax.broadcasted_iota(jnp.int32, (Q, KB), 1)
        cand = jnp.where(rmin == m, rjdx_ref[...] * KB + c,
                         jnp.int32(2**31 - 1))
        idx_ref[...] = jnp.min(cand, axis=1, keepdims=True)


def _nearest_index(queries, keys, label_confi):
    K = keys.shape[0]
    nblk = K // KB
    lab3d = label_confi.reshape(nblk, 1, KB).astype(jnp.int32)
    idx = pl.pallas_call(
        _argmin_body,
        grid=(nblk,),
        in_specs=[
            pl.BlockSpec((Q, D), lambda j: (0, 0)),
            pl.BlockSpec((KB, D), lambda j: (j, 0)),
            pl.BlockSpec((1, 1, KB), lambda j: (j, 0, 0)),
        ],
        out_specs=pl.BlockSpec((Q, 1), lambda j: (0, 0)),
        out_shape=jax.ShapeDtypeStruct((Q, 1), jnp.int32),
        scratch_shapes=[
            pltpu.VMEM((Q, D), jnp.bfloat16),
            pltpu.VMEM((Q, KB), jnp.float32),
            pltpu.VMEM((Q, KB), jnp.int32),
        ],
    )(queries, keys, lab3d)
    return idx.reshape(Q)


def _make_sc_gather(V, B, Dm):
    NC, NS = 2, 16
    NW = NC * NS
    b_per_w = B // NW
    mesh = plsc.VectorSubcoreMesh(core_axis_name="c", subcore_axis_name="s")

    @functools.partial(
        pl.kernel,
        mesh=mesh,
        out_type=jax.ShapeDtypeStruct((B, Dm), jnp.float32),
        scratch_types=[
            pltpu.VMEM((b_per_w,), jnp.int32),
            pltpu.VMEM((b_per_w, Dm), jnp.float32),
            pltpu.SemaphoreType.DMA,
        ],
    )
    def gather_rows(idx_hbm, table_hbm, out_hbm, idx_v, rows_v, sem):
        wid = lax.axis_index("s") * NC + lax.axis_index("c")
        base = wid * b_per_w
        pltpu.sync_copy(idx_hbm.at[pl.ds(base, b_per_w)], idx_v)
        pltpu.async_copy(table_hbm.at[idx_v], rows_v, sem).wait()
        pltpu.sync_copy(rows_v, out_hbm.at[pl.ds(base, b_per_w)])

    return gather_rows


def kernel(queries, keys, label_confi):
    nearest_idx = _nearest_index(queries, keys, label_confi)
    gather = _make_sc_gather(keys.shape[0], Q, D)
    return gather(nearest_idx, keys)
